# baseline (speedup 1.0000x reference)
import functools

import jax
import jax.numpy as jnp
from jax import lax
from jax.experimental import pallas as pl
from jax.experimental.pallas import tpu as pltpu
from jax.experimental.pallas import tpu_sc as plsc

_NC = 2
_NS = 16
_NW = _NC * _NS
_L = 16


def _mm_body(x_ref, w_ref, b_ref, o_ref):
    o_ref[...] = (
        jnp.dot(x_ref[...], w_ref[...], preferred_element_type=jnp.float32)
        + b_ref[...]
    )


def _transform(x_flat, Ws, bs):
    BN, D = x_flat.shape
    BLK = 2000
    return pl.pallas_call(
        _mm_body,
        grid=(BN // BLK,),
        in_specs=[
            pl.BlockSpec((BLK, D), lambda i: (i, 0)),
            pl.BlockSpec((D, D), lambda i: (0, 0)),
            pl.BlockSpec((1, D), lambda i: (0, 0)),
        ],
        out_specs=pl.BlockSpec((BLK, D), lambda i: (i, 0)),
        out_shape=jax.ShapeDtypeStruct((BN, D), jnp.float32),
    )(x_flat, Ws, bs)


def _make_gather_sum(BN, D, K, N):
    CH = 8
    CHK = CH * K
    ngroups = BN // CH
    base_g, extra = divmod(ngroups, _NW)
    ngmax = base_g + (extra > 0)
    NB = 6
    assert BN % CH == 0 and N % CH == 0 and D % _L == 0 and base_g >= NB

    mesh = plsc.VectorSubcoreMesh(
        core_axis_name="c", subcore_axis_name="s", num_cores=_NC,
        num_subcores=_NS)

    @functools.partial(
        pl.kernel,
        mesh=mesh,
        out_type=jax.ShapeDtypeStruct((BN, D), jnp.float32),
        scratch_types=[
            pltpu.VMEM((ngmax * CHK,), jnp.int32),
            [pltpu.VMEM((CHK, D), jnp.float32) for _ in range(NB)],
            [pltpu.VMEM((CH, D), jnp.float32) for _ in range(NB)],
            [pltpu.SemaphoreType.DMA for _ in range(NB)],
            [pltpu.SemaphoreType.DMA for _ in range(NB)],
        ],
    )
    def gather_sum(y_hbm, gidx_hbm, out_hbm, idx_all, rows, outs, sg, so):
        wid = lax.axis_index("s") * _NC + lax.axis_index("c")
        g0 = wid * base_g + lax.min(wid, extra)
        ng = base_g + jnp.where(wid < extra, 1, 0)
        ibase = g0 * CHK

        pltpu.sync_copy(gidx_hbm.at[pl.ds(ibase, base_g * CHK)],
                        idx_all.at[pl.ds(0, base_g * CHK)])

        @pl.when(wid < extra)
        def _tail():
            pltpu.sync_copy(gidx_hbm.at[pl.ds(ibase + base_g * CHK, CHK)],
                            idx_all.at[pl.ds(base_g * CHK, CHK)])

        @pl.loop(0, ng)
        def _off(c):
            boff = ((g0 + c) * CH // N) * N
            for j in range(CHK // _L):
                sl = pl.ds(c * CHK + j * _L, _L)
                idx_all[sl] = idx_all[sl] + boff

        def start_gather(c, bi):
            pltpu.async_copy(
                y_hbm.at[idx_all.at[pl.ds(c * CHK, CHK)]], rows[bi], sg[bi])

        def wait_gather(bi):
            pltpu.make_async_copy(
                y_hbm.at[idx_all.at[pl.ds(0, CHK)]], rows[bi], sg[bi]).wait()

        def wait_out(bi):
            pltpu.make_async_copy(
                outs[bi], out_hbm.at[pl.ds(0, CH)], so[bi]).wait()

        for _b in range(NB):
            start_gather(_b, _b)

        @pl.loop(0, (ngmax + NB - 1) // NB)
        def _pair(p):
            for bi in range(NB):
                cc = p * NB + bi

                @pl.when(cc < ng)
                def _do():
                    wait_gather(bi)

                    @pl.when(cc >= NB)
                    def _wo():
                        wait_out(bi)

                    @pl.loop(0, CH)
                    def _row(r):
                        nd = D // _L
                        accs = [rows[bi][r * K, pl.ds(dd * _L, _L)]
                                for dd in range(nd)]
                        for kk in range(1, K):
                            for dd in range(nd):
                                accs[dd] = accs[dd] + rows[bi][
                                    r * K + kk, pl.ds(dd * _L, _L)]
                        for dd in range(nd):
                            outs[bi][r, pl.ds(dd * _L, _L)] = accs[dd]

                    pltpu.async_copy(
                        outs[bi], out_hbm.at[pl.ds((g0 + cc) * CH, CH)],
                        so[bi])

                    @pl.when(cc + NB < ng)
                    def _ng():
                        start_gather(cc + NB, bi)

        for bi in range(NB):
            wait_out(bi)

    return gather_sum


def kernel(x, knn_idx, W, b):
    B, N, D = x.shape
    K = knn_idx.shape[-1]
    BN = B * N

    x_flat = x.reshape(BN, D)
    Ws = W.T / K
    bs = (b / K).reshape(1, D)
    y = _transform(x_flat, Ws, bs)

    gidx = knn_idx.reshape(BN * K).astype(jnp.int32)
    out_flat = _make_gather_sum(BN, D, K, N)(y, gidx)
    return out_flat.reshape(B, N, D)


# speedup vs baseline: 1.0081x; 1.0081x over previous
import functools

import jax
import jax.numpy as jnp
from jax import lax
from jax.experimental import pallas as pl
from jax.experimental.pallas import tpu as pltpu
from jax.experimental.pallas import tpu_sc as plsc

_NC = 2
_NS = 16
_NW = _NC * _NS
_L = 16


def _mm_body(x_ref, w_ref, b_ref, o_ref):
    o_ref[...] = (
        jnp.dot(x_ref[...], w_ref[...], preferred_element_type=jnp.float32)
        + b_ref[...]
    )


def _transform(x_flat, Ws, bs):
    BN, D = x_flat.shape
    BLK = 2000
    return pl.pallas_call(
        _mm_body,
        grid=(BN // BLK,),
        in_specs=[
            pl.BlockSpec((BLK, D), lambda i: (i, 0)),
            pl.BlockSpec((D, D), lambda i: (0, 0)),
            pl.BlockSpec((1, D), lambda i: (0, 0)),
        ],
        out_specs=pl.BlockSpec((BLK, D), lambda i: (i, 0)),
        out_shape=jax.ShapeDtypeStruct((BN, D), jnp.float32),
    )(x_flat, Ws, bs)


def _make_gather_sum(BN, D, K, N):
    CH = 8
    CHK = CH * K
    ngroups = BN // CH
    base_g, extra = divmod(ngroups, _NW)
    ngmax = base_g + (extra > 0)
    NB = 4
    assert BN % CH == 0 and N % CH == 0 and D % _L == 0 and base_g >= NB

    mesh = plsc.VectorSubcoreMesh(
        core_axis_name="c", subcore_axis_name="s", num_cores=_NC,
        num_subcores=_NS)

    @functools.partial(
        pl.kernel,
        mesh=mesh,
        out_type=jax.ShapeDtypeStruct((BN, D), jnp.float32),
        scratch_types=[
            pltpu.VMEM((ngmax * CHK,), jnp.int32),
            [pltpu.VMEM((CHK, D), jnp.float32) for _ in range(NB)],
            [pltpu.VMEM((CH, D), jnp.float32) for _ in range(NB)],
            [pltpu.SemaphoreType.DMA for _ in range(NB)],
            [pltpu.SemaphoreType.DMA for _ in range(NB)],
        ],
    )
    def gather_sum(y_hbm, gidx_hbm, out_hbm, idx_all, rows, outs, sg, so):
        wid = lax.axis_index("s") * _NC + lax.axis_index("c")
        g0 = wid * base_g + lax.min(wid, extra)
        ng = base_g + jnp.where(wid < extra, 1, 0)
        ibase = g0 * CHK

        pltpu.sync_copy(gidx_hbm.at[pl.ds(ibase, base_g * CHK)],
                        idx_all.at[pl.ds(0, base_g * CHK)])

        @pl.when(wid < extra)
        def _tail():
            pltpu.sync_copy(gidx_hbm.at[pl.ds(ibase + base_g * CHK, CHK)],
                            idx_all.at[pl.ds(base_g * CHK, CHK)])

        @pl.loop(0, ng)
        def _off(c):
            boff = ((g0 + c) * CH // N) * N
            for j in range(CHK // _L):
                sl = pl.ds(c * CHK + j * _L, _L)
                idx_all[sl] = idx_all[sl] + boff

        def start_gather(c, bi):
            pltpu.async_copy(
                y_hbm.at[idx_all.at[pl.ds(c * CHK, CHK)]], rows[bi], sg[bi])

        def wait_gather(bi):
            pltpu.make_async_copy(
                y_hbm.at[idx_all.at[pl.ds(0, CHK)]], rows[bi], sg[bi]).wait()

        def wait_out(bi):
            pltpu.make_async_copy(
                outs[bi], out_hbm.at[pl.ds(0, CH)], so[bi]).wait()

        for _b in range(NB):
            start_gather(_b, _b)

        @pl.loop(0, (ngmax + NB - 1) // NB)
        def _pair(p):
            for bi in range(NB):
                cc = p * NB + bi

                @pl.when(cc < ng)
                def _do():
                    wait_gather(bi)

                    @pl.when(cc >= NB)
                    def _wo():
                        wait_out(bi)

                    @pl.loop(0, CH)
                    def _row(r):
                        nd = D // _L
                        accs = [rows[bi][r * K, pl.ds(dd * _L, _L)]
                                for dd in range(nd)]
                        for kk in range(1, K):
                            for dd in range(nd):
                                accs[dd] = accs[dd] + rows[bi][
                                    r * K + kk, pl.ds(dd * _L, _L)]
                        for dd in range(nd):
                            outs[bi][r, pl.ds(dd * _L, _L)] = accs[dd]

                    pltpu.async_copy(
                        outs[bi], out_hbm.at[pl.ds((g0 + cc) * CH, CH)],
                        so[bi])

                    @pl.when(cc + NB < ng)
                    def _ng():
                        start_gather(cc + NB, bi)

        for bi in range(NB):
            wait_out(bi)

    return gather_sum


def kernel(x, knn_idx, W, b):
    B, N, D = x.shape
    K = knn_idx.shape[-1]
    BN = B * N

    x_flat = x.reshape(BN, D)
    Ws = W.T / K
    bs = (b / K).reshape(1, D)
    y = _transform(x_flat, Ws, bs)

    gidx = knn_idx.reshape(BN * K).astype(jnp.int32)
    out_flat = _make_gather_sum(BN, D, K, N)(y, gidx)
    return out_flat.reshape(B, N, D)
